# baseline (speedup 1.0000x reference)
import jax
import jax.numpy as jnp
from jax.experimental import pallas as pl

SEQ_LEN = 2048
BT_SIZE = 4
D_MODEL = 4096
ROWS = SEQ_LEN * BT_SIZE


def _concat_kernel(x_ref, p_ref, o_ref):
    o_ref[:, :D_MODEL] = x_ref[...]
    o_ref[:, D_MODEL:] = p_ref[...]


def kernel(inputs, pos_table):
    R = 256
    x = inputs.reshape(ROWS, D_MODEL)
    out = pl.pallas_call(
        _concat_kernel,
        grid=(ROWS // R,),
        in_specs=[
            pl.BlockSpec((R, D_MODEL), lambda i: (i, 0)),
            pl.BlockSpec((R, 1), lambda i: (i % (SEQ_LEN // R), 0)),
        ],
        out_specs=pl.BlockSpec((R, D_MODEL + 1), lambda i: (i, 0)),
        out_shape=jax.ShapeDtypeStruct((ROWS, D_MODEL + 1), jnp.float32),
    )(x, pos_table)
    return out.reshape(BT_SIZE, SEQ_LEN, D_MODEL + 1)
